# baseline (speedup 1.0000x reference)
import functools

import jax
import jax.numpy as jnp
from jax import lax
from jax.experimental import pallas as pl
from jax.experimental.pallas import tpu as pltpu
from jax.experimental.pallas import tpu_sc as plsc

_N = 10000
_E = 320000
_D = 128

_NC = 2
_NS = 16
_K = 72
_C0 = 140
_C1 = 140
_CMAX = max(_C0, _C1)
_EPAD = _NS * (_C0 + _C1) * _K
_NPAD = _N + 8
_STRIPE = 624
_TAIL0 = _NS * _STRIPE
_TAIL = _NPAD - _TAIL0


def _make_sc_agg(with_cnt: bool):
    mesh = plsc.VectorSubcoreMesh(
        core_axis_name="c", subcore_axis_name="s", num_cores=_NC,
        num_subcores=_NS)

    part = jax.ShapeDtypeStruct((_N, _D), jnp.float32)
    cntp = jax.ShapeDtypeStruct((_NPAD,), jnp.float32)
    out_type = (part, part, cntp, cntp) if with_cnt else (part, part)

    scratch = [
        pltpu.VMEM((_CMAX * _K,), jnp.int32),
        pltpu.VMEM((_CMAX * _K,), jnp.int32),
        pltpu.VMEM((_K, _D), jnp.float32),
        pltpu.VMEM((_K, _D), jnp.float32),
        pltpu.VMEM_SHARED((_NPAD, _D), jnp.float32),
        pltpu.SemaphoreType.DMA,
        pltpu.SemaphoreType.DMA,
    ]
    if with_cnt:
        scratch += [
            pltpu.VMEM((128,), jnp.float32),
            pltpu.VMEM_SHARED((_NPAD,), jnp.float32),
        ]

    @functools.partial(pl.kernel, mesh=mesh, out_type=out_type,
                       scratch_types=scratch)
    def sc_agg(feat_hbm, src_hbm, dst_hbm, z2_hbm, z1_hbm, *rest):
        if with_cnt:
            (out0, out1, cnt0, cnt1, src_v, dst_v, rows0, rows1, acc,
             sem0, sem1, ones_v, cnt_acc) = rest
        else:
            out0, out1, src_v, dst_v, rows0, rows1, acc, sem0, sem1 = rest
        c = lax.axis_index("c")
        s = lax.axis_index("s")

        if with_cnt:
            for i in range(8):
                ones_v[pl.ds(i * 16, 16)] = jnp.ones((16,), jnp.float32)

        row0 = pl.multiple_of(s * _STRIPE, 8)
        pltpu.sync_copy(z2_hbm.at[pl.ds(row0, _STRIPE)],
                        acc.at[pl.ds(row0, _STRIPE)])

        @pl.when(s == _NS - 1)
        def _():
            pltpu.sync_copy(z2_hbm.at[pl.ds(_TAIL0, _TAIL)],
                            acc.at[pl.ds(_TAIL0, _TAIL)])

        if with_cnt:
            @pl.when(s == 0)
            def _():
                pltpu.sync_copy(z1_hbm, cnt_acc)

        def tile_work(cc, echunk0):
            e0 = pl.multiple_of(echunk0 * _K, 8)
            n_e = cc * _K
            pltpu.sync_copy(src_hbm.at[pl.ds(e0, n_e)],
                            src_v.at[pl.ds(0, n_e)])
            pltpu.sync_copy(dst_hbm.at[pl.ds(e0, n_e)],
                            dst_v.at[pl.ds(0, n_e)])

            pltpu.async_copy(feat_hbm.at[src_v.at[pl.ds(0, _K)]], rows0, sem0)
            pltpu.async_copy(feat_hbm.at[src_v.at[pl.ds(_K, _K)]], rows1,
                             sem1)
            plsc.subcore_barrier()

            def body(i, carry):
                jo = 2 * i
                for b, rows, sem in ((0, rows0, sem0), (1, rows1, sem1)):
                    j = jo + b
                    off = pl.multiple_of(j * _K, 8)
                    pltpu.make_async_copy(
                        feat_hbm.at[src_v.at[pl.ds(off, _K)]], rows,
                        sem).wait()
                    pltpu.sync_copy(rows,
                                    acc.at[dst_v.at[pl.ds(off, _K)]],
                                    add=True)
                    if with_cnt:
                        pltpu.sync_copy(ones_v.at[pl.ds(0, _K)],
                                        cnt_acc.at[dst_v.at[pl.ds(off, _K)]],
                                        add=True)

                    @pl.when(j + 2 < cc)
                    def _():
                        off2 = pl.multiple_of((j + 2) * _K, 8)
                        pltpu.async_copy(
                            feat_hbm.at[src_v.at[pl.ds(off2, _K)]], rows,
                            sem)
                return carry

            lax.fori_loop(0, cc // 2, body, 0)

        @pl.when(c == 0)
        def _():
            tile_work(_C0, s * _C0)

        @pl.when(c == 1)
        def _():
            tile_work(_C1, _NS * _C0 + s * _C1)

        plsc.subcore_barrier()

        for cc_, out in ((0, out0), (1, out1)):
            @pl.when(c == cc_)
            def _(out=out):
                pltpu.sync_copy(acc.at[pl.ds(row0, _STRIPE)],
                                out.at[pl.ds(row0, _STRIPE)])

                @pl.when(s == _NS - 1)
                def _():
                    pltpu.sync_copy(acc.at[pl.ds(_TAIL0, _N - _TAIL0)],
                                    out.at[pl.ds(_TAIL0, _N - _TAIL0)])

        if with_cnt:
            for cc_, cnt in ((0, cnt0), (1, cnt1)):
                @pl.when(jnp.logical_and(c == cc_, s == 0))
                def _(cnt=cnt):
                    pltpu.sync_copy(cnt_acc, cnt)

    return sc_agg


_sc_agg_cnt = _make_sc_agg(True)
_sc_agg = _make_sc_agg(False)


def _tc_layer_body(relu, p0_ref, p1_ref, c0_ref, c1_ref, x_ref, wl_ref,
                   bl_ref, wr_ref, out_ref):
    p = p0_ref[...] + p1_ref[...]
    cnt = c0_ref[...] + c1_ref[...]
    inv = 1.0 / jnp.maximum(cnt, 1.0)
    agg = p * inv[:, None]
    y = (jnp.dot(agg, wl_ref[...], preferred_element_type=jnp.float32,
                 precision=lax.Precision.HIGHEST)
         + bl_ref[...]
         + jnp.dot(x_ref[...], wr_ref[...], preferred_element_type=jnp.float32,
                   precision=lax.Precision.HIGHEST))
    if relu:
        y = jnp.maximum(y, 0.0)
    out_ref[...] = y


def _tc_layer(p0, p1, c0, c1, x, w_l, b_l, w_r, relu):
    return pl.pallas_call(
        functools.partial(_tc_layer_body, relu),
        out_shape=jax.ShapeDtypeStruct((_N, _D), jnp.float32),
    )(p0, p1, c0, c1, x, w_l, b_l, w_r)


def kernel(x, edge_index, W_l1, b_l1, W_r1, W_l2, b_l2, W_r2):
    npad = _EPAD - _E
    src = jnp.concatenate(
        [edge_index[0], jnp.zeros((npad,), jnp.int32)])
    dst = jnp.concatenate(
        [edge_index[1], jnp.full((npad,), _N, jnp.int32)])
    z2 = jnp.zeros((_NPAD, _D), jnp.float32)
    z1 = jnp.zeros((_NPAD,), jnp.float32)

    p0, p1, c0, c1 = _sc_agg_cnt(x, src, dst, z2, z1)
    c0 = c0[:_N]
    c1 = c1[:_N]
    h = _tc_layer(p0, p1, c0, c1, x, W_l1, b_l1.reshape(1, _D), W_r1, True)
    q0, q1 = _sc_agg(h, src, dst, z2, z1)
    out = _tc_layer(q0, q1, c0, c1, h, W_l2, b_l2.reshape(1, _D), W_r2, False)
    return out


# speedup vs baseline: 1.0359x; 1.0359x over previous
import functools

import jax
import jax.numpy as jnp
from jax import lax
from jax.experimental import pallas as pl
from jax.experimental.pallas import tpu as pltpu
from jax.experimental.pallas import tpu_sc as plsc

_N = 10000
_E = 320000
_D = 128

_NC = 2
_NS = 16
_K = 112
_C0 = 92
_C1 = 88
_CMAX = max(_C0, _C1)
_EPAD = _NS * (_C0 + _C1) * _K
_NPAD = _N + 8
_STRIPE = 624
_TAIL0 = _NS * _STRIPE
_TAIL = _NPAD - _TAIL0


def _make_sc_agg(with_cnt: bool):
    mesh = plsc.VectorSubcoreMesh(
        core_axis_name="c", subcore_axis_name="s", num_cores=_NC,
        num_subcores=_NS)

    part = jax.ShapeDtypeStruct((_N, _D), jnp.float32)
    cntp = jax.ShapeDtypeStruct((_NPAD,), jnp.float32)
    out_type = (part, part, cntp, cntp) if with_cnt else (part, part)

    scratch = [
        pltpu.VMEM((_CMAX * _K,), jnp.int32),
        pltpu.VMEM((_CMAX * _K,), jnp.int32),
        pltpu.VMEM((_K, _D), jnp.float32),
        pltpu.VMEM((_K, _D), jnp.float32),
        pltpu.VMEM_SHARED((_NPAD, _D), jnp.float32),
        pltpu.SemaphoreType.DMA,
        pltpu.SemaphoreType.DMA,
    ]
    if with_cnt:
        scratch += [
            pltpu.VMEM((128,), jnp.float32),
            pltpu.VMEM_SHARED((_NPAD,), jnp.float32),
        ]

    @functools.partial(pl.kernel, mesh=mesh, out_type=out_type,
                       scratch_types=scratch)
    def sc_agg(feat_hbm, src_hbm, dst_hbm, z2_hbm, z1_hbm, *rest):
        if with_cnt:
            (out0, out1, cnt0, cnt1, src_v, dst_v, rows0, rows1, acc,
             sem0, sem1, ones_v, cnt_acc) = rest
        else:
            out0, out1, src_v, dst_v, rows0, rows1, acc, sem0, sem1 = rest
        c = lax.axis_index("c")
        s = lax.axis_index("s")

        if with_cnt:
            for i in range(8):
                ones_v[pl.ds(i * 16, 16)] = jnp.ones((16,), jnp.float32)

        row0 = pl.multiple_of(s * _STRIPE, 8)
        pltpu.sync_copy(z2_hbm.at[pl.ds(row0, _STRIPE)],
                        acc.at[pl.ds(row0, _STRIPE)])

        @pl.when(s == _NS - 1)
        def _():
            pltpu.sync_copy(z2_hbm.at[pl.ds(_TAIL0, _TAIL)],
                            acc.at[pl.ds(_TAIL0, _TAIL)])

        if with_cnt:
            @pl.when(s == 0)
            def _():
                pltpu.sync_copy(z1_hbm, cnt_acc)

        def tile_work(cc, echunk0):
            e0 = pl.multiple_of(echunk0 * _K, 8)
            n_e = cc * _K
            pltpu.sync_copy(src_hbm.at[pl.ds(e0, n_e)],
                            src_v.at[pl.ds(0, n_e)])
            pltpu.sync_copy(dst_hbm.at[pl.ds(e0, n_e)],
                            dst_v.at[pl.ds(0, n_e)])

            pltpu.async_copy(feat_hbm.at[src_v.at[pl.ds(0, _K)]], rows0, sem0)
            pltpu.async_copy(feat_hbm.at[src_v.at[pl.ds(_K, _K)]], rows1,
                             sem1)
            plsc.subcore_barrier()

            def body(i, carry):
                jo = 2 * i
                for b, rows, sem in ((0, rows0, sem0), (1, rows1, sem1)):
                    j = jo + b
                    off = pl.multiple_of(j * _K, 8)
                    pltpu.make_async_copy(
                        feat_hbm.at[src_v.at[pl.ds(off, _K)]], rows,
                        sem).wait()
                    pltpu.sync_copy(rows,
                                    acc.at[dst_v.at[pl.ds(off, _K)]],
                                    add=True)
                    if with_cnt:
                        pltpu.sync_copy(ones_v.at[pl.ds(0, _K)],
                                        cnt_acc.at[dst_v.at[pl.ds(off, _K)]],
                                        add=True)

                    @pl.when(j + 2 < cc)
                    def _():
                        off2 = pl.multiple_of((j + 2) * _K, 8)
                        pltpu.async_copy(
                            feat_hbm.at[src_v.at[pl.ds(off2, _K)]], rows,
                            sem)
                return carry

            lax.fori_loop(0, cc // 2, body, 0)

        @pl.when(c == 0)
        def _():
            tile_work(_C0, s * _C0)

        @pl.when(c == 1)
        def _():
            tile_work(_C1, _NS * _C0 + s * _C1)

        plsc.subcore_barrier()

        for cc_, out in ((0, out0), (1, out1)):
            @pl.when(c == cc_)
            def _(out=out):
                pltpu.sync_copy(acc.at[pl.ds(row0, _STRIPE)],
                                out.at[pl.ds(row0, _STRIPE)])

                @pl.when(s == _NS - 1)
                def _():
                    pltpu.sync_copy(acc.at[pl.ds(_TAIL0, _N - _TAIL0)],
                                    out.at[pl.ds(_TAIL0, _N - _TAIL0)])

        if with_cnt:
            for cc_, cnt in ((0, cnt0), (1, cnt1)):
                @pl.when(jnp.logical_and(c == cc_, s == 0))
                def _(cnt=cnt):
                    pltpu.sync_copy(cnt_acc, cnt)

    return sc_agg


_sc_agg_cnt = _make_sc_agg(True)
_sc_agg = _make_sc_agg(False)


def _tc_layer_body(relu, p0_ref, p1_ref, c0_ref, c1_ref, x_ref, wl_ref,
                   bl_ref, wr_ref, out_ref):
    p = p0_ref[...] + p1_ref[...]
    cnt = c0_ref[...] + c1_ref[...]
    inv = 1.0 / jnp.maximum(cnt, 1.0)
    agg = p * inv[:, None]
    y = (jnp.dot(agg, wl_ref[...], preferred_element_type=jnp.float32,
                 precision=lax.Precision.HIGHEST)
         + bl_ref[...]
         + jnp.dot(x_ref[...], wr_ref[...], preferred_element_type=jnp.float32,
                   precision=lax.Precision.HIGHEST))
    if relu:
        y = jnp.maximum(y, 0.0)
    out_ref[...] = y


def _tc_layer(p0, p1, c0, c1, x, w_l, b_l, w_r, relu):
    return pl.pallas_call(
        functools.partial(_tc_layer_body, relu),
        out_shape=jax.ShapeDtypeStruct((_N, _D), jnp.float32),
    )(p0, p1, c0, c1, x, w_l, b_l, w_r)


def kernel(x, edge_index, W_l1, b_l1, W_r1, W_l2, b_l2, W_r2):
    npad = _EPAD - _E
    src = jnp.concatenate(
        [edge_index[0], jnp.zeros((npad,), jnp.int32)])
    dst = jnp.concatenate(
        [edge_index[1], jnp.full((npad,), _N, jnp.int32)])
    z2 = jnp.zeros((_NPAD, _D), jnp.float32)
    z1 = jnp.zeros((_NPAD,), jnp.float32)

    p0, p1, c0, c1 = _sc_agg_cnt(x, src, dst, z2, z1)
    c0 = c0[:_N]
    c1 = c1[:_N]
    h = _tc_layer(p0, p1, c0, c1, x, W_l1, b_l1.reshape(1, _D), W_r1, True)
    q0, q1 = _sc_agg(h, src, dst, z2, z1)
    out = _tc_layer(q0, q1, c0, c1, h, W_l2, b_l2.reshape(1, _D), W_r2, False)
    return out


# speedup vs baseline: 1.6748x; 1.6167x over previous
import functools

import jax
import jax.numpy as jnp
from jax import lax
from jax.experimental import pallas as pl
from jax.experimental.pallas import tpu as pltpu
from jax.experimental.pallas import tpu_sc as plsc

_N = 10000
_E = 320000
_D = 128

_NC = 2
_NS = 16
_K = 72
_C0 = 140
_C1 = 138
_CMAX = max(_C0, _C1)
_EPAD = _NS * (_C0 + _C1) * _K
_NPAD = _N + 8
_STRIPE = 624
_TAIL0 = _NS * _STRIPE
_TAIL = _NPAD - _TAIL0


def _make_sc_agg(with_cnt: bool):
    mesh = plsc.VectorSubcoreMesh(
        core_axis_name="c", subcore_axis_name="s", num_cores=_NC,
        num_subcores=_NS)

    part = jax.ShapeDtypeStruct((_N, _D), jnp.float32)
    cntp = jax.ShapeDtypeStruct((_NPAD,), jnp.float32)
    out_type = (part, part, cntp, cntp) if with_cnt else (part, part)

    scratch = [
        pltpu.VMEM((_CMAX * _K,), jnp.int32),
        pltpu.VMEM((_CMAX * _K,), jnp.int32),
        pltpu.VMEM((_K, _D), jnp.float32),
        pltpu.VMEM((_K, _D), jnp.float32),
        pltpu.VMEM_SHARED((_NPAD, _D), jnp.float32),
        pltpu.SemaphoreType.DMA,
        pltpu.SemaphoreType.DMA,
    ]
    if with_cnt:
        scratch += [
            pltpu.VMEM((128,), jnp.float32),
            pltpu.VMEM_SHARED((_NPAD,), jnp.float32),
        ]

    @functools.partial(pl.kernel, mesh=mesh, out_type=out_type,
                       scratch_types=scratch)
    def sc_agg(feat_hbm, src_hbm, dst_hbm, z2_hbm, z1_hbm, *rest):
        if with_cnt:
            (out0, out1, cnt0, cnt1, src_v, dst_v, rows0, rows1, acc,
             sem0, sem1, ones_v, cnt_acc) = rest
        else:
            out0, out1, src_v, dst_v, rows0, rows1, acc, sem0, sem1 = rest
        c = lax.axis_index("c")
        s = lax.axis_index("s")

        if with_cnt:
            for i in range(8):
                ones_v[pl.ds(i * 16, 16)] = jnp.ones((16,), jnp.float32)

        row0 = pl.multiple_of(s * _STRIPE, 8)
        pltpu.sync_copy(z2_hbm.at[pl.ds(row0, _STRIPE)],
                        acc.at[pl.ds(row0, _STRIPE)])

        @pl.when(s == _NS - 1)
        def _():
            pltpu.sync_copy(z2_hbm.at[pl.ds(_TAIL0, _TAIL)],
                            acc.at[pl.ds(_TAIL0, _TAIL)])

        if with_cnt:
            @pl.when(s == 0)
            def _():
                pltpu.sync_copy(z1_hbm, cnt_acc)

        def tile_work(cc, echunk0):
            e0 = pl.multiple_of(echunk0 * _K, 8)
            n_e = cc * _K
            pltpu.sync_copy(src_hbm.at[pl.ds(e0, n_e)],
                            src_v.at[pl.ds(0, n_e)])
            pltpu.sync_copy(dst_hbm.at[pl.ds(e0, n_e)],
                            dst_v.at[pl.ds(0, n_e)])

            pltpu.async_copy(feat_hbm.at[src_v.at[pl.ds(0, _K)]], rows0, sem0)
            pltpu.async_copy(feat_hbm.at[src_v.at[pl.ds(_K, _K)]], rows1,
                             sem1)
            plsc.subcore_barrier()

            def body(i, carry):
                jo = 2 * i
                for b, rows, sem in ((0, rows0, sem0), (1, rows1, sem1)):
                    j = jo + b
                    off = pl.multiple_of(j * _K, 8)
                    pltpu.make_async_copy(
                        feat_hbm.at[src_v.at[pl.ds(off, _K)]], rows,
                        sem).wait()
                    pltpu.sync_copy(rows,
                                    acc.at[dst_v.at[pl.ds(off, _K)]],
                                    add=True)
                    if with_cnt:
                        pltpu.sync_copy(ones_v.at[pl.ds(0, _K)],
                                        cnt_acc.at[dst_v.at[pl.ds(off, _K)]],
                                        add=True)

                    @pl.when(j + 2 < cc)
                    def _():
                        off2 = pl.multiple_of((j + 2) * _K, 8)
                        pltpu.async_copy(
                            feat_hbm.at[src_v.at[pl.ds(off2, _K)]], rows,
                            sem)
                return carry

            lax.fori_loop(0, cc // 2, body, 0)

        @pl.when(c == 0)
        def _():
            tile_work(_C0, s * _C0)

        @pl.when(c == 1)
        def _():
            tile_work(_C1, _NS * _C0 + s * _C1)

        plsc.subcore_barrier()

        for cc_, out in ((0, out0), (1, out1)):
            @pl.when(c == cc_)
            def _(out=out):
                pltpu.sync_copy(acc.at[pl.ds(row0, _STRIPE)],
                                out.at[pl.ds(row0, _STRIPE)])

                @pl.when(s == _NS - 1)
                def _():
                    pltpu.sync_copy(acc.at[pl.ds(_TAIL0, _N - _TAIL0)],
                                    out.at[pl.ds(_TAIL0, _N - _TAIL0)])

        if with_cnt:
            for cc_, cnt in ((0, cnt0), (1, cnt1)):
                @pl.when(jnp.logical_and(c == cc_, s == 0))
                def _(cnt=cnt):
                    pltpu.sync_copy(cnt_acc, cnt)

    return sc_agg


_sc_agg_cnt = _make_sc_agg(True)
_sc_agg = _make_sc_agg(False)


def _tc_layer_body(relu, p0_ref, p1_ref, c0_ref, c1_ref, x_ref, wl_ref,
                   bl_ref, wr_ref, out_ref):
    p = p0_ref[...] + p1_ref[...]
    cnt = c0_ref[...] + c1_ref[...]
    inv = 1.0 / jnp.maximum(cnt, 1.0)
    agg = p * inv[:, None]
    y = (jnp.dot(agg, wl_ref[...], preferred_element_type=jnp.float32,
                 precision=lax.Precision.HIGHEST)
         + bl_ref[...]
         + jnp.dot(x_ref[...], wr_ref[...], preferred_element_type=jnp.float32,
                   precision=lax.Precision.HIGHEST))
    if relu:
        y = jnp.maximum(y, 0.0)
    out_ref[...] = y


def _tc_layer(p0, p1, c0, c1, x, w_l, b_l, w_r, relu):
    return pl.pallas_call(
        functools.partial(_tc_layer_body, relu),
        out_shape=jax.ShapeDtypeStruct((_N, _D), jnp.float32),
    )(p0, p1, c0, c1, x, w_l, b_l, w_r)


def kernel(x, edge_index, W_l1, b_l1, W_r1, W_l2, b_l2, W_r2):
    npad = _EPAD - _E
    src = jnp.concatenate(
        [edge_index[0], jnp.zeros((npad,), jnp.int32)])
    dst = jnp.concatenate(
        [edge_index[1], jnp.full((npad,), _N, jnp.int32)])
    z2 = jnp.zeros((_NPAD, _D), jnp.float32)
    z1 = jnp.zeros((_NPAD,), jnp.float32)

    p0, p1, c0, c1 = _sc_agg_cnt(x, src, dst, z2, z1)
    c0 = c0[:_N]
    c1 = c1[:_N]
    h = _tc_layer(p0, p1, c0, c1, x, W_l1, b_l1.reshape(1, _D), W_r1, True)
    q0, q1 = _sc_agg(h, src, dst, z2, z1)
    out = _tc_layer(q0, q1, c0, c1, h, W_l2, b_l2.reshape(1, _D), W_r2, False)
    return out
